# baseline (speedup 1.0000x reference)
import functools

import jax
import jax.numpy as jnp
from jax import lax
from jax.experimental import pallas as pl
from jax.experimental.pallas import tpu as pltpu
from jax.experimental.pallas import tpu_sc as plsc

T = 4096
D = 1024
E = 8
NRE = 7
H = 4096
ALPHA = 2.0

BT = 256
NBR = 22
NBS = T // BT
NB = NBR + NBS
TP = NBR * BT
TPX = TP + T
BH = 256
NH = H // BH

NC, NS = 2, 16
NW = NC * NS
TW = T // NW
NBID = 48


RBT = 512


def _router_body(x_ref, w_ref, eid_ref, scale_ref):
    lg = jnp.dot(x_ref[...], w_ref[...], preferred_element_type=jnp.float32)
    m = jnp.max(lg, axis=1, keepdims=True)
    col = lax.broadcasted_iota(jnp.int32, lg.shape, 1)
    amax = jnp.min(jnp.where(lg == m, col, E), axis=1)
    p = 1.0 / jnp.sum(jnp.exp(lg - m), axis=1)
    routed = amax < NRE
    eid_ref[0, 0, :] = jnp.where(routed, amax, 0)
    scale_ref[0, 0, :] = jnp.where(routed, ALPHA * p, 0.0)


def _router(x, router_w):
    ntb = T // RBT
    eid, scale = pl.pallas_call(
        _router_body,
        grid=(ntb,),
        in_specs=[
            pl.BlockSpec((RBT, D), lambda i: (i, 0)),
            pl.BlockSpec((D, E), lambda i: (0, 0)),
        ],
        out_specs=[
            pl.BlockSpec((1, 1, RBT), lambda i: (i, 0, 0)),
            pl.BlockSpec((1, 1, RBT), lambda i: (i, 0, 0)),
        ],
        out_shape=[
            jax.ShapeDtypeStruct((ntb, 1, RBT), jnp.int32),
            jax.ShapeDtypeStruct((ntb, 1, RBT), jnp.float32),
        ],
    )(x, router_w)
    return eid.reshape(T), scale.reshape(T)


def _sort_scatter(eid, x):
    mesh = plsc.VectorSubcoreMesh(core_axis_name="c", subcore_axis_name="s",
                                  num_cores=NC, num_subcores=NS)

    @functools.partial(
        pl.kernel,
        mesh=mesh,
        compiler_params=pltpu.CompilerParams(needs_layout_passes=False),
        out_type=(
            jax.ShapeDtypeStruct((T,), jnp.int32),
            jax.ShapeDtypeStruct((NBID,), jnp.int32),
            jax.ShapeDtypeStruct((TPX, D), jnp.float32),
        ),
        scratch_types=[
            pltpu.VMEM((T,), jnp.int32),
            pltpu.VMEM((NBID,), jnp.int32),
            pltpu.VMEM((TW,), jnp.int32),
            pltpu.VMEM((16,), jnp.int32),
            pltpu.VMEM((16, D), jnp.float32),
            pltpu.SemaphoreType.DMA,
        ],
    )
    def k(eid_hbm, x_hbm, dest_hbm, beid_hbm, xs_hbm,
          eidall, beidv, destv, idxv, xrow, sem):
        wid = lax.axis_index("s") * NC + lax.axis_index("c")
        base = wid * TW
        lanes = lax.iota(jnp.int32, 16)

        pltpu.sync_copy(eid_hbm, eidall)

        myfirst = wid * (TW // 16)

        def hbody(jj, carry):
            tot, pref = carry
            v = eidall[pl.ds(jj * 16, 16)]
            contrib = jnp.zeros((16,), jnp.int32)
            for e in range(NRE):
                c = plsc.cumsum(jnp.where(v == e, 1, 0))[15]
                contrib = contrib + jnp.where(lanes == e, c, 0)
            return (tot + contrib,
                    pref + jnp.where(jj < myfirst, contrib, 0))

        tot, pref = lax.fori_loop(
            0, T // 16, hbody,
            (jnp.zeros((16,), jnp.int32), jnp.zeros((16,), jnp.int32)))
        nblk = (tot + (BT - 1)) >> 8
        blk_incl = plsc.cumsum(nblk)
        blk_excl = blk_incl - nblk
        wbase = blk_excl * BT + pref

        for c2 in range(NBID // 16):
            bv = lax.iota(jnp.int32, 16) + c2 * 16
            acc = jnp.zeros((16,), jnp.int32)
            for e in range(NRE):
                s = blk_excl[e]
                n = nblk[e]
                acc = jnp.where((bv >= s) & (bv < s + n), e, acc)
            acc = jnp.where(bv >= NBR, NRE, acc)
            beidv[pl.ds(c2 * 16, 16)] = acc

        @pl.when(wid == 0)
        def _():
            pltpu.sync_copy(beidv, beid_hbm)

        run = wbase
        for j in range(TW // 16):
            v = eidall[pl.ds(base + j * 16, 16)]
            dest16 = jnp.zeros((16,), jnp.int32)
            for e in range(NRE):
                m = v == e
                csum = plsc.cumsum(jnp.where(m, 1, 0))
                rank = csum - 1
                r_es = run[e]
                dest16 = jnp.where(m, r_es + rank, dest16)
                run = run + jnp.where(lanes == e, csum[15], 0)
            destv[pl.ds(j * 16, 16)] = dest16
            idxv[...] = dest16
            pltpu.sync_copy(x_hbm.at[pl.ds(base + j * 16, 16)], xrow)
            pltpu.async_copy(xrow, xs_hbm.at[idxv], sem).wait()

        pltpu.sync_copy(destv, dest_hbm.at[pl.ds(base, TW)])
        pltpu.sync_copy(x_hbm.at[pl.ds(base, TW)],
                        xs_hbm.at[pl.ds(TP + base, TW)])

    return k(eid, x)


def _mlp_body(beid_s, xs_ref, eu, eg, ed, su, sg, sd, out_ref, acc_ref):
    h = pl.program_id(0)
    b = pl.program_id(1)
    e = beid_s[b]
    shared = e == NRE
    up_w = jnp.where(shared, su[...], eu[0])
    gt_w = jnp.where(shared, sg[...], eg[0])
    dn_w = jnp.where(shared, sd[...], ed[0])
    xb = xs_ref[...]
    up = jnp.dot(xb, up_w, preferred_element_type=jnp.float32)
    gt = jnp.dot(xb, gt_w, preferred_element_type=jnp.float32)
    a = up * (gt * jax.nn.sigmoid(gt))
    part = jnp.dot(a, dn_w, preferred_element_type=jnp.float32)
    sl = pl.ds(b * BT, BT)

    @pl.when(h == 0)
    def _():
        acc_ref[sl, :] = part

    @pl.when(h > 0)
    def _():
        acc_ref[sl, :] = acc_ref[sl, :] + part

    @pl.when(h == NH - 1)
    def _():
        out_ref[...] = acc_ref[sl, :]


def _grouped_mlp(beid, xs, eu, eg, ed, su, sg, sd):
    grid_spec = pltpu.PrefetchScalarGridSpec(
        num_scalar_prefetch=1,
        grid=(NH, NB),
        in_specs=[
            pl.BlockSpec((BT, D), lambda h, b, beid_s: (b, 0)),
            pl.BlockSpec((1, D, BH),
                         lambda h, b, beid_s: (jnp.minimum(beid_s[b], NRE - 1), 0, h)),
            pl.BlockSpec((1, D, BH),
                         lambda h, b, beid_s: (jnp.minimum(beid_s[b], NRE - 1), 0, h)),
            pl.BlockSpec((1, BH, D),
                         lambda h, b, beid_s: (jnp.minimum(beid_s[b], NRE - 1), h, 0)),
            pl.BlockSpec((D, BH), lambda h, b, beid_s: (0, h)),
            pl.BlockSpec((D, BH), lambda h, b, beid_s: (0, h)),
            pl.BlockSpec((BH, D), lambda h, b, beid_s: (h, 0)),
        ],
        out_specs=pl.BlockSpec((BT, D), lambda h, b, beid_s: (b, 0)),
        scratch_shapes=[pltpu.VMEM((NB * BT, D), jnp.float32)],
    )
    return pl.pallas_call(
        _mlp_body,
        grid_spec=grid_spec,
        out_shape=jax.ShapeDtypeStruct((TPX, D), jnp.float32),
        compiler_params=pltpu.CompilerParams(
            dimension_semantics=("arbitrary", "arbitrary")),
    )(beid, xs, eu, eg, ed, su, sg, sd)


def _combine(outs, dest, scale):
    mesh = plsc.VectorSubcoreMesh(core_axis_name="c", subcore_axis_name="s",
                                  num_cores=NC, num_subcores=NS)

    @functools.partial(
        pl.kernel,
        mesh=mesh,
        compiler_params=pltpu.CompilerParams(needs_layout_passes=False),
        out_type=jax.ShapeDtypeStruct((T, D), jnp.float32),
        scratch_types=[
            pltpu.VMEM((TW,), jnp.int32),
            pltpu.VMEM((TW,), jnp.float32),
            pltpu.VMEM((16,), jnp.int32),
            pltpu.VMEM((16, D), jnp.float32),
            pltpu.VMEM((16, D), jnp.float32),
            pltpu.SemaphoreType.DMA,
        ],
    )
    def k(outs_hbm, dest_hbm, scale_hbm, out_hbm, destv, scalev, idxv, g, s,
          sem):
        wid = lax.axis_index("s") * NC + lax.axis_index("c")
        base = wid * TW
        pltpu.sync_copy(dest_hbm.at[pl.ds(base, TW)], destv)
        pltpu.sync_copy(scale_hbm.at[pl.ds(base, TW)], scalev)
        for j in range(TW // 16):
            idxv[...] = destv[pl.ds(j * 16, 16)]
            pltpu.async_copy(outs_hbm.at[idxv], g, sem).wait()
            pltpu.sync_copy(outs_hbm.at[pl.ds(TP + base + j * 16, 16)], s)
            sv = scalev[pl.ds(j * 16, 16)]
            for r in range(16):
                sc = sv[r]

                def cbody(c, _):
                    cs = pl.ds(c * 16, 16)
                    g[r, cs] = g[r, cs] * sc + s[r, cs]
                    return 0

                lax.fori_loop(0, D // 16, cbody, 0)
            pltpu.sync_copy(g, out_hbm.at[pl.ds(base + j * 16, 16)])

    return k(outs, dest, scale)


def kernel(x, router_w, expert_up, expert_gate, expert_down,
           shared_up, shared_gate, shared_down):
    eid, scale = _router(x, router_w)
    dest, beid, xs = _sort_scatter(eid, x)
    outs = _grouped_mlp(beid, xs, expert_up, expert_gate, expert_down,
                        shared_up, shared_gate, shared_down)
    return _combine(outs, dest, scale)


# speedup vs baseline: 2.7514x; 2.7514x over previous
import functools

import jax
import jax.numpy as jnp
from jax import lax
from jax.experimental import pallas as pl
from jax.experimental.pallas import tpu as pltpu
from jax.experimental.pallas import tpu_sc as plsc

T = 4096
D = 1024
E = 8
NRE = 7
H = 4096
ALPHA = 2.0

BT = 256
NBR = 22
TP = NBR * BT
BH = 512
NH = H // BH

NC, NS = 2, 16
NW = NC * NS
TW = T // NW
NBID = 32
SG = 128


RBT = 512
NTB = T // RBT


def _router_body(x_ref, w_ref, eid_ref, scale_ref, cnt_ref):
    lg = jnp.dot(x_ref[...], w_ref[...], preferred_element_type=jnp.float32)
    m = jnp.max(lg, axis=1, keepdims=True)
    col = lax.broadcasted_iota(jnp.int32, lg.shape, 1)
    amax = jnp.min(jnp.where(lg == m, col, E), axis=1)
    p = 1.0 / jnp.sum(jnp.exp(lg - m), axis=1)
    routed = amax < NRE
    eid = jnp.where(routed, amax, 0)
    eid_ref[0, 0, :] = eid
    scale_ref[0, 0, :] = jnp.where(routed, ALPHA * p, 0.0)
    cols16 = lax.broadcasted_iota(jnp.int32, (RBT, 16), 1)
    oh = (eid[:, None] == cols16).astype(jnp.int32)
    for c2 in range(RBT // TW):
        cnt_ref[0, c2, :] = jnp.sum(oh[c2 * TW:(c2 + 1) * TW], axis=0)


def _router(x, router_w):
    eid, scale, cnt = pl.pallas_call(
        _router_body,
        grid=(NTB,),
        in_specs=[
            pl.BlockSpec((RBT, D), lambda i: (i, 0)),
            pl.BlockSpec((D, E), lambda i: (0, 0)),
        ],
        out_specs=[
            pl.BlockSpec((1, 1, RBT), lambda i: (i, 0, 0)),
            pl.BlockSpec((1, 1, RBT), lambda i: (i, 0, 0)),
            pl.BlockSpec((1, RBT // TW, 16), lambda i: (i, 0, 0)),
        ],
        out_shape=[
            jax.ShapeDtypeStruct((NTB, 1, RBT), jnp.int32),
            jax.ShapeDtypeStruct((NTB, 1, RBT), jnp.float32),
            jax.ShapeDtypeStruct((NTB, RBT // TW, 16), jnp.int32),
        ],
    )(x, router_w)
    return eid.reshape(T), scale.reshape(T), cnt.reshape(NW * 16)


def _sort_scatter(eid, scale, cnt, x):
    mesh = plsc.VectorSubcoreMesh(core_axis_name="c", subcore_axis_name="s",
                                  num_cores=NC, num_subcores=NS)

    @functools.partial(
        pl.kernel,
        mesh=mesh,
        compiler_params=pltpu.CompilerParams(needs_layout_passes=False),
        out_type=(
            jax.ShapeDtypeStruct((T,), jnp.int32),
            jax.ShapeDtypeStruct((NBID,), jnp.int32),
            jax.ShapeDtypeStruct((TP, D), jnp.float32),
            jax.ShapeDtypeStruct((TP, SG), jnp.float32),
        ),
        scratch_types=[
            pltpu.VMEM((NW * 16,), jnp.int32),
            pltpu.VMEM((TW,), jnp.int32),
            pltpu.VMEM((TW,), jnp.float32),
            pltpu.VMEM((NBID,), jnp.int32),
            pltpu.VMEM((TW,), jnp.int32),
            pltpu.VMEM((64,), jnp.int32),
            pltpu.VMEM((64,), jnp.int32),
            pltpu.VMEM((64, D), jnp.float32),
            pltpu.VMEM((64, SG), jnp.float32),
            pltpu.SemaphoreType.DMA,
        ],
    )
    def k(eid_hbm, scale_hbm, cnt_hbm, x_hbm,
          dest_hbm, beid_hbm, xs_hbm, ss_hbm,
          cntv, eidv, scalev, beidv, destv, idxA, idxB, xbuf, sbuf, sem):
        wid = lax.axis_index("s") * NC + lax.axis_index("c")
        base = wid * TW
        lanes = lax.iota(jnp.int32, 16)

        pltpu.sync_copy(cnt_hbm, cntv)
        pltpu.sync_copy(eid_hbm.at[pl.ds(base, TW)], eidv)
        pltpu.sync_copy(scale_hbm.at[pl.ds(base, TW)], scalev)

        tot = jnp.zeros((16,), jnp.int32)
        pref = jnp.zeros((16,), jnp.int32)
        for w in range(NW):
            row = cntv[pl.ds(w * 16, 16)]
            tot = tot + row
            pref = pref + jnp.where(w < wid, row, jnp.zeros((16,), jnp.int32))
        nblk = (tot + (BT - 1)) >> 8
        blk_incl = plsc.cumsum(nblk)
        blk_excl = blk_incl - nblk
        wbase = blk_excl * BT + pref

        for c2 in range(NBID // 16):
            bv = lax.iota(jnp.int32, 16) + c2 * 16
            acc = jnp.zeros((16,), jnp.int32)
            for e in range(NRE):
                s = blk_excl[e]
                n = nblk[e]
                acc = jnp.where((bv >= s) & (bv < s + n), e, acc)
            beidv[pl.ds(c2 * 16, 16)] = acc

        @pl.when(wid == 0)
        def _():
            pltpu.sync_copy(beidv, beid_hbm)

        run = wbase
        for j in range(TW // 16):
            v = eidv[pl.ds(j * 16, 16)]
            dest16 = jnp.zeros((16,), jnp.int32)
            for e in range(NRE):
                m = v == e
                csum = plsc.cumsum(jnp.where(m, 1, 0))
                r_es = run[e]
                dest16 = jnp.where(m, r_es + csum - 1, dest16)
                run = run + jnp.where(lanes == e, csum[15], 0)
            destv[pl.ds(j * 16, 16)] = dest16
            half = idxA if j < 4 else idxB
            half[pl.ds((j % 4) * 16, 16)] = dest16
        pltpu.sync_copy(destv, dest_hbm.at[pl.ds(base, TW)])

        for half, idx in ((0, idxA), (1, idxB)):
            hb = base + half * 64
            for q in range(4):
                s16 = scalev[pl.ds(half * 64 + q * 16, 16)]
                for r in range(16):
                    sbuf[q * 16 + r, pl.ds(0, 16)] = jnp.full(
                        (16,), s16[r], jnp.float32)
            pltpu.sync_copy(x_hbm.at[pl.ds(hb, 64)], xbuf)
            pltpu.async_copy(xbuf, xs_hbm.at[idx], sem).wait()
            pltpu.async_copy(sbuf, ss_hbm.at[idx], sem).wait()

    return k(eid, scale, cnt, x)


def _mlp_body(beid_s, xs_ref, ss_ref, eu, eg, ed, su, sg, sd, out_ref,
              acc_ref):
    h = pl.program_id(0)
    b = pl.program_id(1)
    xb = xs_ref[...]
    up = jnp.dot(xb, eu[0], preferred_element_type=jnp.float32)
    gt = jnp.dot(xb, eg[0], preferred_element_type=jnp.float32)
    a = up * (gt * jax.nn.sigmoid(gt))
    part_r = jnp.dot(a, ed[0], preferred_element_type=jnp.float32)
    ups = jnp.dot(xb, su[...], preferred_element_type=jnp.float32)
    gts = jnp.dot(xb, sg[...], preferred_element_type=jnp.float32)
    a_s = ups * (gts * jax.nn.sigmoid(gts))
    part_s = jnp.dot(a_s, sd[...], preferred_element_type=jnp.float32)
    sc = ss_ref[:, 0:1]
    part = sc * part_r + part_s
    sl = pl.ds(b * BT, BT)

    @pl.when(h == 0)
    def _():
        acc_ref[sl, :] = part

    @pl.when(h > 0)
    def _():
        acc_ref[sl, :] = acc_ref[sl, :] + part

    @pl.when(h == NH - 1)
    def _():
        out_ref[...] = acc_ref[sl, :]


def _grouped_mlp(beid, xs, ss, eu, eg, ed, su, sg, sd):
    grid_spec = pltpu.PrefetchScalarGridSpec(
        num_scalar_prefetch=1,
        grid=(NH, NBR),
        in_specs=[
            pl.BlockSpec((BT, D), lambda h, b, beid_s: (b, 0)),
            pl.BlockSpec((BT, SG), lambda h, b, beid_s: (b, 0)),
            pl.BlockSpec((1, D, BH), lambda h, b, beid_s: (beid_s[b], 0, h)),
            pl.BlockSpec((1, D, BH), lambda h, b, beid_s: (beid_s[b], 0, h)),
            pl.BlockSpec((1, BH, D), lambda h, b, beid_s: (beid_s[b], h, 0)),
            pl.BlockSpec((D, BH), lambda h, b, beid_s: (0, h)),
            pl.BlockSpec((D, BH), lambda h, b, beid_s: (0, h)),
            pl.BlockSpec((BH, D), lambda h, b, beid_s: (h, 0)),
        ],
        out_specs=pl.BlockSpec((BT, D), lambda h, b, beid_s: (b, 0)),
        scratch_shapes=[pltpu.VMEM((TP, D), jnp.float32)],
    )
    return pl.pallas_call(
        _mlp_body,
        grid_spec=grid_spec,
        out_shape=jax.ShapeDtypeStruct((TP, D), jnp.float32),
        compiler_params=pltpu.CompilerParams(
            dimension_semantics=("arbitrary", "arbitrary")),
    )(beid, xs, ss, eu, eg, ed, su, sg, sd)


def _gather_out(rows, dest):
    mesh = plsc.VectorSubcoreMesh(core_axis_name="c", subcore_axis_name="s",
                                  num_cores=NC, num_subcores=NS)

    @functools.partial(
        pl.kernel,
        mesh=mesh,
        compiler_params=pltpu.CompilerParams(needs_layout_passes=False),
        out_type=jax.ShapeDtypeStruct((T, D), jnp.float32),
        scratch_types=[
            pltpu.VMEM((TW,), jnp.int32),
            pltpu.VMEM((64,), jnp.int32),
            pltpu.VMEM((64, D), jnp.float32),
            pltpu.SemaphoreType.DMA,
        ],
    )
    def k(rows_hbm, dest_hbm, out_hbm, destv, idx, buf, sem):
        wid = lax.axis_index("s") * NC + lax.axis_index("c")
        base = wid * TW
        pltpu.sync_copy(dest_hbm.at[pl.ds(base, TW)], destv)
        for half in range(2):
            for q in range(4):
                idx[pl.ds(q * 16, 16)] = destv[pl.ds(half * 64 + q * 16, 16)]
            pltpu.async_copy(rows_hbm.at[idx], buf, sem).wait()
            pltpu.sync_copy(buf, out_hbm.at[pl.ds(base + half * 64, 64)])

    return k(rows, dest)


def kernel(x, router_w, expert_up, expert_gate, expert_down,
           shared_up, shared_gate, shared_down):
    eid, scale, cnt = _router(x, router_w)
    dest, beid, xs, ss = _sort_scatter(eid, scale, cnt, x)
    rows = _grouped_mlp(beid, xs, ss, expert_up, expert_gate, expert_down,
                        shared_up, shared_gate, shared_down)
    return _gather_out(rows, dest)
